# baseline (speedup 1.0000x reference)
import jax
import jax.numpy as jnp
from jax.experimental import pallas as pl
from jax.experimental.pallas import tpu as pltpu

D = 768
H = 12
HD = 64
E = 64
K = 2
FFN = 64
SCALE = 0.1
NEG = -1e30

_F32 = jnp.float32
_BF16 = jnp.bfloat16


def _ln_qkv_kernel(x_ref, lnw_ref, lnb_ref, w_ref, b_ref, o_ref):
    ys = []
    for b in range(2):
        x = x_ref[:, b * D:(b + 1) * D]
        m = jnp.mean(x, axis=1, keepdims=True)
        v = jnp.mean((x - m) ** 2, axis=1, keepdims=True)
        xn = (x - m) / jnp.sqrt(v + 1e-5) * lnw_ref[...] + lnb_ref[...]
        y = jnp.dot(xn.astype(_BF16), w_ref[...], preferred_element_type=_F32)
        ys.append((y + b_ref[...]).astype(_BF16))
    o_ref[...] = jnp.concatenate(ys, axis=1)


def _attn_kernel(q_ref, k_ref, v_ref, x_ref, w_ref, b_ref, o_ref):
    hp = pl.program_id(2)
    q2 = q_ref[...]
    k2 = k_ref[...]
    v2 = v_ref[...]
    CK = 512
    S_FULL = k2.shape[0]
    outs = []
    for h in range(2):
        q = q2[:, h * HD:(h + 1) * HD]
        acc = None
        den = None
        for c in range(S_FULL // CK):
            kc = k2[c * CK:(c + 1) * CK, h * HD:(h + 1) * HD]
            vc = v2[c * CK:(c + 1) * CK, h * HD:(h + 1) * HD]
            s = jax.lax.dot_general(q, kc, (((1,), (1,)), ((), ())),
                                    preferred_element_type=_F32)
            e = jnp.exp(s)
            d = jnp.sum(e, axis=1, keepdims=True)
            o = jnp.dot(e.astype(_BF16), vc, preferred_element_type=_F32)
            acc = o if acc is None else acc + o
            den = d if den is None else den + d
        outs.append((acc * (1.0 / den)).astype(_BF16))
    o2 = jnp.concatenate(outs, axis=1)
    po = jnp.dot(o2, w_ref[...], preferred_element_type=_F32)

    @pl.when(hp == 0)
    def _():
        o_ref[...] = x_ref[...] + b_ref[...]

    o_ref[...] += po


def _router_kernel(x_ref, rw_ref, idx_ref, gate_ref):
    row = jax.lax.broadcasted_iota(jnp.int32, (8, D), 0)
    c0 = x_ref[0:1, 0:D]
    c1 = x_ref[0:1, D:2 * D]
    xc = jnp.where(row == 0, c0, jnp.where(row == 1, c1, 0.0))
    logits = jnp.dot(xc.astype(_BF16), rw_ref[...].astype(_BF16),
                     preferred_element_type=_F32)
    col = jax.lax.broadcasted_iota(jnp.int32, logits.shape, 1)
    m1 = jnp.max(logits, axis=1, keepdims=True)
    i1 = jnp.min(jnp.where(logits == m1, col, E), axis=1, keepdims=True)
    l2 = jnp.where(col == i1, NEG, logits)
    m2 = jnp.max(l2, axis=1, keepdims=True)
    i2 = jnp.min(jnp.where(l2 == m2, col, E), axis=1, keepdims=True)
    g1 = 1.0 / (1.0 + jnp.exp(m2 - m1))
    g2 = 1.0 - g1
    ocol = jax.lax.broadcasted_iota(jnp.int32, (8, 128), 1)
    idx_ref[...] = jnp.where(ocol == 0, i1, jnp.where(ocol == 1, i2, 0))
    gate_ref[...] = jnp.where(ocol == 0, g1, jnp.where(ocol == 1, g2, 0.0))


def _mlp_moe_kernel(idx_ref, g_ref, x_ref, lnw_ref, lnb_ref, wfc_ref, bfc_ref,
                    wpr_ref, bpr_ref, dw0_ref, db0_ref, uw0_ref, ub0_ref,
                    dw1_ref, db1_ref, uw1_ref, ub1_ref, o_ref):
    b = pl.program_id(0)
    x = x_ref[...]
    m = jnp.mean(x, axis=1, keepdims=True)
    v = jnp.mean((x - m) ** 2, axis=1, keepdims=True)
    xn = (x - m) / jnp.sqrt(v + 1e-5) * lnw_ref[...] + lnb_ref[...]
    h = jnp.dot(xn.astype(_BF16), wfc_ref[...], preferred_element_type=_F32)
    h = h + bfc_ref[...]
    h = h * jax.nn.sigmoid(1.702 * h)
    y = jnp.dot(h.astype(_BF16), wpr_ref[...], preferred_element_type=_F32)
    out = y + bpr_ref[...] + x

    xh = x.astype(_BF16)
    for kk, (dw, db, uw, ub) in enumerate(
            ((dw0_ref, db0_ref, uw0_ref, ub0_ref),
             (dw1_ref, db1_ref, uw1_ref, ub1_ref))):
        g = g_ref[b, kk] * SCALE
        hh = jnp.dot(xh, dw[0].astype(_BF16),
                     preferred_element_type=_F32) + db[0]
        hh = jnp.maximum(hh, 0.0)
        up = jnp.dot(hh.astype(_BF16), uw[0].astype(_BF16),
                     preferred_element_type=_F32)
        out = out + g * (up + ub[0])
    o_ref[...] = out


def kernel(x, in_proj_w, in_proj_b, out_proj_w, out_proj_b, ln1_w, ln1_b,
           ln2_w, ln2_b, c_fc_w, c_fc_b, c_proj_w, c_proj_b, router,
           down_w, down_b, up_w, up_b):
    S, B, _ = x.shape
    BS = 1024
    BQ = 1024
    nS = S // BS

    xr = x.reshape(S, B * D)
    qscale = jnp.concatenate([jnp.full((D,), 0.125, _F32),
                              jnp.ones((2 * D,), _F32)])
    w_in = (in_proj_w.T * qscale).astype(_BF16)
    in_proj_b = in_proj_b * qscale
    w_out = out_proj_w.T.astype(_BF16)
    w_fc = c_fc_w.T.astype(_BF16)
    w_pr = c_proj_w.T.astype(_BF16)
    db2 = down_b.reshape(E, 1, FFN)
    ub2 = up_b.reshape(E, 1, D)

    seq = ("arbitrary",)
    TD = 3 * D

    qkv = pl.pallas_call(
        _ln_qkv_kernel,
        grid=(nS,),
        in_specs=[
            pl.BlockSpec((BS, B * D), lambda i: (i, 0)),
            pl.BlockSpec((1, D), lambda i: (0, 0)),
            pl.BlockSpec((1, D), lambda i: (0, 0)),
            pl.BlockSpec((D, TD), lambda i: (0, 0)),
            pl.BlockSpec((1, TD), lambda i: (0, 0)),
        ],
        out_specs=pl.BlockSpec((BS, B * TD), lambda i: (i, 0)),
        out_shape=jax.ShapeDtypeStruct((S, B * TD), _BF16),
        compiler_params=pltpu.CompilerParams(dimension_semantics=seq),
    )(xr, ln1_w.reshape(1, D), ln1_b.reshape(1, D), w_in,
      in_proj_b.reshape(1, TD))

    HP = H // 2
    NB = TD // 128
    x1 = pl.pallas_call(
        _attn_kernel,
        grid=(B, S // BQ, HP),
        in_specs=[
            pl.BlockSpec((BQ, 2 * HD), lambda b, i, h: (i, b * NB + h)),
            pl.BlockSpec((S, 2 * HD), lambda b, i, h: (0, b * NB + HP + h)),
            pl.BlockSpec((S, 2 * HD),
                         lambda b, i, h: (0, b * NB + 2 * HP + h)),
            pl.BlockSpec((BQ, D), lambda b, i, h: (i, b)),
            pl.BlockSpec((2 * HD, D), lambda b, i, h: (h, 0)),
            pl.BlockSpec((1, D), lambda b, i, h: (0, 0)),
        ],
        out_specs=pl.BlockSpec((BQ, D), lambda b, i, h: (i, b)),
        out_shape=jax.ShapeDtypeStruct((S, B * D), _F32),
        compiler_params=pltpu.CompilerParams(
            dimension_semantics=seq * 3),
    )(qkv, qkv, qkv, xr, w_out, out_proj_b.reshape(1, D))

    idx_p, gate_p = pl.pallas_call(
        _router_kernel,
        grid=(1,),
        in_specs=[
            pl.BlockSpec((8, B * D), lambda i: (0, 0)),
            pl.BlockSpec((D, E), lambda i: (0, 0)),
        ],
        out_specs=[
            pl.BlockSpec((8, 128), lambda i: (0, 0)),
            pl.BlockSpec((8, 128), lambda i: (0, 0)),
        ],
        out_shape=[
            jax.ShapeDtypeStruct((8, 128), jnp.int32),
            jax.ShapeDtypeStruct((8, 128), _F32),
        ],
        compiler_params=pltpu.CompilerParams(dimension_semantics=seq),
    )(x1, router)

    grid_spec = pltpu.PrefetchScalarGridSpec(
        num_scalar_prefetch=2,
        grid=(B, nS),
        in_specs=[
            pl.BlockSpec((BS, D), lambda b, i, ir, gr: (i, b)),
            pl.BlockSpec((1, D), lambda b, i, ir, gr: (0, 0)),
            pl.BlockSpec((1, D), lambda b, i, ir, gr: (0, 0)),
            pl.BlockSpec((D, 4 * D), lambda b, i, ir, gr: (0, 0)),
            pl.BlockSpec((1, 4 * D), lambda b, i, ir, gr: (0, 0)),
            pl.BlockSpec((4 * D, D), lambda b, i, ir, gr: (0, 0)),
            pl.BlockSpec((1, D), lambda b, i, ir, gr: (0, 0)),
            pl.BlockSpec((1, D, FFN), lambda b, i, ir, gr: (ir[b, 0], 0, 0)),
            pl.BlockSpec((1, 1, FFN), lambda b, i, ir, gr: (ir[b, 0], 0, 0)),
            pl.BlockSpec((1, FFN, D), lambda b, i, ir, gr: (ir[b, 0], 0, 0)),
            pl.BlockSpec((1, 1, D), lambda b, i, ir, gr: (ir[b, 0], 0, 0)),
            pl.BlockSpec((1, D, FFN), lambda b, i, ir, gr: (ir[b, 1], 0, 0)),
            pl.BlockSpec((1, 1, FFN), lambda b, i, ir, gr: (ir[b, 1], 0, 0)),
            pl.BlockSpec((1, FFN, D), lambda b, i, ir, gr: (ir[b, 1], 0, 0)),
            pl.BlockSpec((1, 1, D), lambda b, i, ir, gr: (ir[b, 1], 0, 0)),
        ],
        out_specs=pl.BlockSpec((BS, D), lambda b, i, ir, gr: (i, b)),
    )
    out_f = pl.pallas_call(
        _mlp_moe_kernel,
        grid_spec=grid_spec,
        out_shape=jax.ShapeDtypeStruct((S, B * D), _F32),
        compiler_params=pltpu.CompilerParams(
            dimension_semantics=seq * 2),
    )(idx_p, gate_p, x1, ln2_w.reshape(1, D), ln2_b.reshape(1, D), w_fc,
      c_fc_b.reshape(1, 4 * D), w_pr, c_proj_b.reshape(1, D),
      down_w, db2, up_w, ub2, down_w, db2, up_w, ub2)

    return out_f.reshape(S, B, D)


# speedup vs baseline: 1.0024x; 1.0024x over previous
import jax
import jax.numpy as jnp
from jax.experimental import pallas as pl
from jax.experimental.pallas import tpu as pltpu

D = 768
H = 12
HD = 64
E = 64
K = 2
FFN = 64
SCALE = 0.1
NEG = -1e30

_F32 = jnp.float32
_BF16 = jnp.bfloat16


def _ln_qkv_kernel(x_ref, lnw_ref, lnb_ref, w_ref, b_ref, o_ref):
    ys = []
    for b in range(2):
        x = x_ref[:, b * D:(b + 1) * D]
        m = jnp.mean(x, axis=1, keepdims=True)
        v = jnp.mean((x - m) ** 2, axis=1, keepdims=True)
        xn = (x - m) / jnp.sqrt(v + 1e-5) * lnw_ref[...] + lnb_ref[...]
        y = jnp.dot(xn.astype(_BF16), w_ref[...], preferred_element_type=_F32)
        ys.append((y + b_ref[...]).astype(_BF16))
    o_ref[...] = jnp.concatenate(ys, axis=1)


def _attn_kernel(q_ref, k_ref, v_ref, x_ref, w_ref, b_ref, o_ref):
    hp = pl.program_id(2)
    q2 = q_ref[...]
    k2 = k_ref[...]
    v2 = v_ref[...]
    CK = 512
    S_FULL = k2.shape[0]
    outs = []
    for h in range(2):
        q = q2[:, h * HD:(h + 1) * HD]
        acc = None
        den = None
        for c in range(S_FULL // CK):
            kc = k2[c * CK:(c + 1) * CK, h * HD:(h + 1) * HD]
            vc = v2[c * CK:(c + 1) * CK, h * HD:(h + 1) * HD]
            s = jax.lax.dot_general(q, kc, (((1,), (1,)), ((), ())),
                                    preferred_element_type=_F32)
            e = jnp.exp(s)
            d = jnp.sum(e, axis=1, keepdims=True)
            o = jnp.dot(e.astype(_BF16), vc, preferred_element_type=_F32)
            acc = o if acc is None else acc + o
            den = d if den is None else den + d
        outs.append((acc * (1.0 / den)).astype(_BF16))
    o2 = jnp.concatenate(outs, axis=1)
    po = jnp.dot(o2, w_ref[...], preferred_element_type=_F32)

    @pl.when(hp == 0)
    def _():
        o_ref[...] = x_ref[...] + b_ref[...]

    o_ref[...] += po


def _router_kernel(x_ref, rw_ref, idx_ref, gate_ref):
    row = jax.lax.broadcasted_iota(jnp.int32, (8, D), 0)
    c0 = x_ref[0:1, 0:D]
    c1 = x_ref[0:1, D:2 * D]
    xc = jnp.where(row == 0, c0, jnp.where(row == 1, c1, 0.0))
    logits = jnp.dot(xc.astype(_BF16), rw_ref[...].astype(_BF16),
                     preferred_element_type=_F32)
    col = jax.lax.broadcasted_iota(jnp.int32, logits.shape, 1)
    m1 = jnp.max(logits, axis=1, keepdims=True)
    i1 = jnp.min(jnp.where(logits == m1, col, E), axis=1, keepdims=True)
    l2 = jnp.where(col == i1, NEG, logits)
    m2 = jnp.max(l2, axis=1, keepdims=True)
    i2 = jnp.min(jnp.where(l2 == m2, col, E), axis=1, keepdims=True)
    g1 = 1.0 / (1.0 + jnp.exp(m2 - m1))
    g2 = 1.0 - g1
    ocol = jax.lax.broadcasted_iota(jnp.int32, (8, 128), 1)
    idx_ref[...] = jnp.where(ocol == 0, i1, jnp.where(ocol == 1, i2, 0))
    gate_ref[...] = jnp.where(ocol == 0, g1, jnp.where(ocol == 1, g2, 0.0))


def _mlp_moe_kernel(idx_ref, g_ref, x_ref, lnw_ref, lnb_ref, wfc_ref, bfc_ref,
                    wpr_ref, bpr_ref, dw0_ref, db0_ref, uw0_ref, ub0_ref,
                    dw1_ref, db1_ref, uw1_ref, ub1_ref, o_ref):
    b = pl.program_id(0)
    x = x_ref[...]
    m = jnp.mean(x, axis=1, keepdims=True)
    v = jnp.mean((x - m) ** 2, axis=1, keepdims=True)
    xn = (x - m) / jnp.sqrt(v + 1e-5) * lnw_ref[...] + lnb_ref[...]
    h = jnp.dot(xn.astype(_BF16), wfc_ref[...], preferred_element_type=_F32)
    h = h + bfc_ref[...]
    h = h * jax.nn.sigmoid(1.702 * h)
    y = jnp.dot(h.astype(_BF16), wpr_ref[...], preferred_element_type=_F32)
    out = y + bpr_ref[...] + x

    xh = x.astype(_BF16)
    for kk, (dw, db, uw, ub) in enumerate(
            ((dw0_ref, db0_ref, uw0_ref, ub0_ref),
             (dw1_ref, db1_ref, uw1_ref, ub1_ref))):
        g = g_ref[b, kk] * SCALE
        hh = jnp.dot(xh, dw[0].astype(_BF16),
                     preferred_element_type=_F32) + db[0]
        hh = jnp.maximum(hh, 0.0)
        up = jnp.dot(hh.astype(_BF16), uw[0].astype(_BF16),
                     preferred_element_type=_F32)
        out = out + g * (up + ub[0])
    o_ref[...] = out


def kernel(x, in_proj_w, in_proj_b, out_proj_w, out_proj_b, ln1_w, ln1_b,
           ln2_w, ln2_b, c_fc_w, c_fc_b, c_proj_w, c_proj_b, router,
           down_w, down_b, up_w, up_b):
    S, B, _ = x.shape
    BS1 = 512
    BS = 512
    BQ = 1024
    nS1 = S // BS1
    nS = S // BS

    xr = x.reshape(S, B * D)
    qscale = jnp.concatenate([jnp.full((D,), 0.125, _F32),
                              jnp.ones((2 * D,), _F32)])
    w_in = (in_proj_w.T * qscale).astype(_BF16)
    in_proj_b = in_proj_b * qscale
    w_out = out_proj_w.T.astype(_BF16)
    w_fc = c_fc_w.T.astype(_BF16)
    w_pr = c_proj_w.T.astype(_BF16)
    db2 = down_b.reshape(E, 1, FFN)
    ub2 = up_b.reshape(E, 1, D)

    seq = ("arbitrary",)
    TD = 3 * D

    qkv = pl.pallas_call(
        _ln_qkv_kernel,
        grid=(nS1,),
        in_specs=[
            pl.BlockSpec((BS1, B * D), lambda i: (i, 0)),
            pl.BlockSpec((1, D), lambda i: (0, 0)),
            pl.BlockSpec((1, D), lambda i: (0, 0)),
            pl.BlockSpec((D, TD), lambda i: (0, 0)),
            pl.BlockSpec((1, TD), lambda i: (0, 0)),
        ],
        out_specs=pl.BlockSpec((BS1, B * TD), lambda i: (i, 0)),
        out_shape=jax.ShapeDtypeStruct((S, B * TD), _BF16),
        compiler_params=pltpu.CompilerParams(dimension_semantics=seq),
    )(xr, ln1_w.reshape(1, D), ln1_b.reshape(1, D), w_in,
      in_proj_b.reshape(1, TD))

    HP = H // 2
    NB = TD // 128
    x1 = pl.pallas_call(
        _attn_kernel,
        grid=(B, S // BQ, HP),
        in_specs=[
            pl.BlockSpec((BQ, 2 * HD), lambda b, i, h: (i, b * NB + h)),
            pl.BlockSpec((S, 2 * HD), lambda b, i, h: (0, b * NB + HP + h)),
            pl.BlockSpec((S, 2 * HD),
                         lambda b, i, h: (0, b * NB + 2 * HP + h)),
            pl.BlockSpec((BQ, D), lambda b, i, h: (i, b)),
            pl.BlockSpec((2 * HD, D), lambda b, i, h: (h, 0)),
            pl.BlockSpec((1, D), lambda b, i, h: (0, 0)),
        ],
        out_specs=pl.BlockSpec((BQ, D), lambda b, i, h: (i, b)),
        out_shape=jax.ShapeDtypeStruct((S, B * D), _F32),
        compiler_params=pltpu.CompilerParams(
            dimension_semantics=seq * 3),
    )(qkv, qkv, qkv, xr, w_out, out_proj_b.reshape(1, D))

    idx_p, gate_p = pl.pallas_call(
        _router_kernel,
        grid=(1,),
        in_specs=[
            pl.BlockSpec((8, B * D), lambda i: (0, 0)),
            pl.BlockSpec((D, E), lambda i: (0, 0)),
        ],
        out_specs=[
            pl.BlockSpec((8, 128), lambda i: (0, 0)),
            pl.BlockSpec((8, 128), lambda i: (0, 0)),
        ],
        out_shape=[
            jax.ShapeDtypeStruct((8, 128), jnp.int32),
            jax.ShapeDtypeStruct((8, 128), _F32),
        ],
        compiler_params=pltpu.CompilerParams(dimension_semantics=seq),
    )(x1, router)

    grid_spec = pltpu.PrefetchScalarGridSpec(
        num_scalar_prefetch=2,
        grid=(B, nS),
        in_specs=[
            pl.BlockSpec((BS, D), lambda b, i, ir, gr: (i, b)),
            pl.BlockSpec((1, D), lambda b, i, ir, gr: (0, 0)),
            pl.BlockSpec((1, D), lambda b, i, ir, gr: (0, 0)),
            pl.BlockSpec((D, 4 * D), lambda b, i, ir, gr: (0, 0)),
            pl.BlockSpec((1, 4 * D), lambda b, i, ir, gr: (0, 0)),
            pl.BlockSpec((4 * D, D), lambda b, i, ir, gr: (0, 0)),
            pl.BlockSpec((1, D), lambda b, i, ir, gr: (0, 0)),
            pl.BlockSpec((1, D, FFN), lambda b, i, ir, gr: (ir[b, 0], 0, 0)),
            pl.BlockSpec((1, 1, FFN), lambda b, i, ir, gr: (ir[b, 0], 0, 0)),
            pl.BlockSpec((1, FFN, D), lambda b, i, ir, gr: (ir[b, 0], 0, 0)),
            pl.BlockSpec((1, 1, D), lambda b, i, ir, gr: (ir[b, 0], 0, 0)),
            pl.BlockSpec((1, D, FFN), lambda b, i, ir, gr: (ir[b, 1], 0, 0)),
            pl.BlockSpec((1, 1, FFN), lambda b, i, ir, gr: (ir[b, 1], 0, 0)),
            pl.BlockSpec((1, FFN, D), lambda b, i, ir, gr: (ir[b, 1], 0, 0)),
            pl.BlockSpec((1, 1, D), lambda b, i, ir, gr: (ir[b, 1], 0, 0)),
        ],
        out_specs=pl.BlockSpec((BS, D), lambda b, i, ir, gr: (i, b)),
    )
    out_f = pl.pallas_call(
        _mlp_moe_kernel,
        grid_spec=grid_spec,
        out_shape=jax.ShapeDtypeStruct((S, B * D), _F32),
        compiler_params=pltpu.CompilerParams(
            dimension_semantics=seq * 2),
    )(idx_p, gate_p, x1, ln2_w.reshape(1, D), ln2_b.reshape(1, D), w_fc,
      c_fc_b.reshape(1, 4 * D), w_pr, c_proj_b.reshape(1, D),
      down_w, db2, up_w, ub2, down_w, db2, up_w, ub2)

    return out_f.reshape(S, B, D)


# speedup vs baseline: 1.0445x; 1.0420x over previous
import jax
import jax.numpy as jnp
from jax.experimental import pallas as pl
from jax.experimental.pallas import tpu as pltpu

D = 768
H = 12
HD = 64
E = 64
K = 2
FFN = 64
SCALE = 0.1
NEG = -1e30

_F32 = jnp.float32
_BF16 = jnp.bfloat16


def _ln_qkv_kernel(x_ref, lnw_ref, lnb_ref, w_ref, b_ref, o_ref):
    x = x_ref[0].astype(_F32)
    m = jnp.mean(x, axis=1, keepdims=True)
    v = jnp.mean((x - m) ** 2, axis=1, keepdims=True)
    xn = (x - m) / jnp.sqrt(v + 1e-5) * lnw_ref[...] + lnb_ref[...]
    y = jnp.dot(xn.astype(_BF16), w_ref[...], preferred_element_type=_F32)
    y = y + b_ref[...]
    o_ref[0] = y.astype(_BF16)


def _attn_kernel(q_ref, k_ref, v_ref, x_ref, w_ref, b_ref, o_ref):
    hp = pl.program_id(2)
    q2 = q_ref[0]
    k2 = k_ref[0]
    v2 = v_ref[0]
    CK = 512
    S_FULL = k2.shape[0]
    outs = []
    for h in range(2):
        q = q2[:, h * HD:(h + 1) * HD]
        acc = None
        den = None
        for c in range(S_FULL // CK):
            kc = k2[c * CK:(c + 1) * CK, h * HD:(h + 1) * HD]
            vc = v2[c * CK:(c + 1) * CK, h * HD:(h + 1) * HD]
            s = jax.lax.dot_general(q, kc, (((1,), (1,)), ((), ())),
                                    preferred_element_type=_F32)
            e = jnp.exp(s)
            d = jnp.sum(e, axis=1, keepdims=True)
            o = jnp.dot(e.astype(_BF16), vc, preferred_element_type=_F32)
            acc = o if acc is None else acc + o
            den = d if den is None else den + d
        outs.append((acc * (1.0 / den)).astype(_BF16))
    o2 = jnp.concatenate(outs, axis=1)
    po = jnp.dot(o2, w_ref[...], preferred_element_type=_F32)

    @pl.when(hp == 0)
    def _():
        o_ref[0] = x_ref[0] + b_ref[...]

    o_ref[0] += po


def _router_kernel(x0_ref, x1_ref, rw_ref, idx_ref, gate_ref):
    row = jax.lax.broadcasted_iota(jnp.int32, (8, D), 0)
    c0 = x0_ref[0][0:1, :]
    c1 = x1_ref[0][0:1, :]
    xc = jnp.where(row == 0, c0, jnp.where(row == 1, c1, 0.0))
    logits = jnp.dot(xc.astype(_BF16), rw_ref[...].astype(_BF16),
                     preferred_element_type=_F32)
    col = jax.lax.broadcasted_iota(jnp.int32, logits.shape, 1)
    m1 = jnp.max(logits, axis=1, keepdims=True)
    i1 = jnp.min(jnp.where(logits == m1, col, E), axis=1, keepdims=True)
    l2 = jnp.where(col == i1, NEG, logits)
    m2 = jnp.max(l2, axis=1, keepdims=True)
    i2 = jnp.min(jnp.where(l2 == m2, col, E), axis=1, keepdims=True)
    g1 = 1.0 / (1.0 + jnp.exp(m2 - m1))
    g2 = 1.0 - g1
    ocol = jax.lax.broadcasted_iota(jnp.int32, (8, 128), 1)
    idx_ref[...] = jnp.where(ocol == 0, i1, jnp.where(ocol == 1, i2, 0))
    gate_ref[...] = jnp.where(ocol == 0, g1, jnp.where(ocol == 1, g2, 0.0))


def _mlp_moe_kernel(idx_ref, g_ref, x_ref, lnw_ref, lnb_ref, wfc_ref, bfc_ref,
                    wpr_ref, bpr_ref, dw_ref, db_ref, uw_ref, ub_ref, o_ref):
    b = pl.program_id(0)
    k = pl.program_id(2)
    x = x_ref[0]

    @pl.when(k == 0)
    def _():
        m = jnp.mean(x, axis=1, keepdims=True)
        v = jnp.mean((x - m) ** 2, axis=1, keepdims=True)
        xn = (x - m) / jnp.sqrt(v + 1e-5) * lnw_ref[...] + lnb_ref[...]
        h = jnp.dot(xn.astype(_BF16), wfc_ref[...],
                    preferred_element_type=_F32)
        h = h + bfc_ref[...]
        h = h * jax.nn.sigmoid(1.702 * h)
        y = jnp.dot(h.astype(_BF16), wpr_ref[...],
                    preferred_element_type=_F32)
        o_ref[0] = y + bpr_ref[...] + x

    g = g_ref[b, k] * SCALE
    xh = x.astype(_BF16)
    hh = jnp.dot(xh, dw_ref[0].astype(_BF16),
                 preferred_element_type=_F32) + db_ref[0]
    hh = jnp.maximum(hh, 0.0)
    up = jnp.dot(hh.astype(_BF16), uw_ref[0].astype(_BF16),
                 preferred_element_type=_F32)
    o_ref[0] += g * (up + ub_ref[0])


def kernel(x, in_proj_w, in_proj_b, out_proj_w, out_proj_b, ln1_w, ln1_b,
           ln2_w, ln2_b, c_fc_w, c_fc_b, c_proj_w, c_proj_b, router,
           down_w, down_b, up_w, up_b):
    S, B, _ = x.shape
    BS = 1024
    BQ = 1024
    nS = S // BS

    xb = jnp.transpose(x, (1, 0, 2))
    qscale = jnp.concatenate([jnp.full((D,), 0.125, _F32),
                              jnp.ones((2 * D,), _F32)])
    w_in = (in_proj_w.T * qscale).astype(_BF16)
    in_proj_b = in_proj_b * qscale
    w_out = out_proj_w.T.astype(_BF16)
    w_fc = c_fc_w.T.astype(_BF16)
    w_pr = c_proj_w.T.astype(_BF16)
    db2 = down_b.reshape(E, 1, FFN)
    ub2 = up_b.reshape(E, 1, D)

    seq = ("arbitrary",)

    qkv = pl.pallas_call(
        _ln_qkv_kernel,
        grid=(B, nS),
        in_specs=[
            pl.BlockSpec((1, BS, D), lambda b, i: (b, i, 0)),
            pl.BlockSpec((1, D), lambda b, i: (0, 0)),
            pl.BlockSpec((1, D), lambda b, i: (0, 0)),
            pl.BlockSpec((D, 3 * D), lambda b, i: (0, 0)),
            pl.BlockSpec((1, 3 * D), lambda b, i: (0, 0)),
        ],
        out_specs=pl.BlockSpec((1, BS, 3 * D), lambda b, i: (b, i, 0)),
        out_shape=jax.ShapeDtypeStruct((B, S, 3 * D), _BF16),
        compiler_params=pltpu.CompilerParams(
            dimension_semantics=seq * 2),
    )(xb, ln1_w.reshape(1, D), ln1_b.reshape(1, D), w_in,
      in_proj_b.reshape(1, 3 * D))

    HP = H // 2
    x1 = pl.pallas_call(
        _attn_kernel,
        grid=(B, S // BQ, HP),
        in_specs=[
            pl.BlockSpec((1, BQ, 2 * HD), lambda b, i, h: (b, i, h)),
            pl.BlockSpec((1, S, 2 * HD), lambda b, i, h: (b, 0, HP + h)),
            pl.BlockSpec((1, S, 2 * HD), lambda b, i, h: (b, 0, 2 * HP + h)),
            pl.BlockSpec((1, BQ, D), lambda b, i, h: (b, i, 0)),
            pl.BlockSpec((2 * HD, D), lambda b, i, h: (h, 0)),
            pl.BlockSpec((1, D), lambda b, i, h: (0, 0)),
        ],
        out_specs=pl.BlockSpec((1, BQ, D), lambda b, i, h: (b, i, 0)),
        out_shape=jax.ShapeDtypeStruct((B, S, D), _F32),
        compiler_params=pltpu.CompilerParams(
            dimension_semantics=seq * 3),
    )(qkv, qkv, qkv, xb, w_out, out_proj_b.reshape(1, D))

    idx_p, gate_p = pl.pallas_call(
        _router_kernel,
        grid=(1,),
        in_specs=[
            pl.BlockSpec((1, 8, D), lambda i: (0, 0, 0)),
            pl.BlockSpec((1, 8, D), lambda i: (1, 0, 0)),
            pl.BlockSpec((D, E), lambda i: (0, 0)),
        ],
        out_specs=[
            pl.BlockSpec((8, 128), lambda i: (0, 0)),
            pl.BlockSpec((8, 128), lambda i: (0, 0)),
        ],
        out_shape=[
            jax.ShapeDtypeStruct((8, 128), jnp.int32),
            jax.ShapeDtypeStruct((8, 128), _F32),
        ],
        compiler_params=pltpu.CompilerParams(dimension_semantics=seq),
    )(x1, x1, router)

    grid_spec = pltpu.PrefetchScalarGridSpec(
        num_scalar_prefetch=2,
        grid=(B, nS, K),
        in_specs=[
            pl.BlockSpec((1, BS, D), lambda b, i, k, ir, gr: (b, i, 0)),
            pl.BlockSpec((1, D), lambda b, i, k, ir, gr: (0, 0)),
            pl.BlockSpec((1, D), lambda b, i, k, ir, gr: (0, 0)),
            pl.BlockSpec((D, 4 * D), lambda b, i, k, ir, gr: (0, 0)),
            pl.BlockSpec((1, 4 * D), lambda b, i, k, ir, gr: (0, 0)),
            pl.BlockSpec((4 * D, D), lambda b, i, k, ir, gr: (0, 0)),
            pl.BlockSpec((1, D), lambda b, i, k, ir, gr: (0, 0)),
            pl.BlockSpec((1, D, FFN),
                         lambda b, i, k, ir, gr: (ir[b, k], 0, 0)),
            pl.BlockSpec((1, 1, FFN),
                         lambda b, i, k, ir, gr: (ir[b, k], 0, 0)),
            pl.BlockSpec((1, FFN, D),
                         lambda b, i, k, ir, gr: (ir[b, k], 0, 0)),
            pl.BlockSpec((1, 1, D),
                         lambda b, i, k, ir, gr: (ir[b, k], 0, 0)),
        ],
        out_specs=pl.BlockSpec(
            (1, BS, D), lambda b, i, k, ir, gr: (b, i, 0)),
    )
    out_b = pl.pallas_call(
        _mlp_moe_kernel,
        grid_spec=grid_spec,
        out_shape=jax.ShapeDtypeStruct((B, S, D), _F32),
        compiler_params=pltpu.CompilerParams(
            dimension_semantics=seq * 3),
    )(idx_p, gate_p, x1, ln2_w.reshape(1, D), ln2_b.reshape(1, D), w_fc,
      c_fc_b.reshape(1, 4 * D), w_pr, c_proj_b.reshape(1, D),
      down_w, db2, up_w, ub2)

    return jnp.transpose(out_b, (1, 0, 2))
